# baseline (speedup 1.0000x reference)
import functools

import jax
import jax.numpy as jnp
from jax import lax
from jax.experimental import pallas as pl
from jax.experimental.pallas import tpu as pltpu
from jax.experimental.pallas import tpu_sc as plsc

NC = 2
NS = 16
L = 16
NW = NC * NS

B = 4096
S = 50
E = 64
O = 128
V = 1000000

PAIR_ROWS = V // 2
ZPAD = 64
PR = PAIR_ROWS + ZPAD
ZROW = PAIR_ROWS

CHUNK_C = 128
N_CHUNKS = V // CHUNK_C
TAIL_C = V - N_CHUNKS * CHUNK_C
PAIR_LOOP = 122

SEQ_PER_W = B // NW


def _sc_prep(tab_t, tail_pairs):
  mesh = plsc.VectorSubcoreMesh(
      core_axis_name="c", subcore_axis_name="s", num_cores=NC, num_subcores=NS)

  @functools.partial(
      pl.kernel,
      out_type=jax.ShapeDtypeStruct((PR, 128), jnp.float32),
      mesh=mesh,
      scratch_types=[
          pltpu.VMEM((E, CHUNK_C), jnp.float32),
          pltpu.VMEM((E, CHUNK_C), jnp.float32),
          pltpu.VMEM((E, 128), jnp.float32),
          pltpu.VMEM((E, 128), jnp.float32),
          pltpu.SemaphoreType.DMA,
          pltpu.SemaphoreType.DMA,
          pltpu.SemaphoreType.DMA,
          pltpu.SemaphoreType.DMA,
      ],
      compiler_params=pltpu.CompilerParams(needs_layout_passes=False),
  )
  def k(tab_hbm, tail_hbm, pairs_hbm, in0, in1, ob0, ob1, si0, si1, so0, so1):
    wid = lax.axis_index("s") * NC + lax.axis_index("c")

    row_vecs = [lax.iota(jnp.int32, L) + (g * L) for g in range(E // L)]

    def zero_out(ob):
      z = jnp.zeros((L,), jnp.float32)

      def zb(r, carry):
        for g in range(8):
          ob[r, pl.ds(g * L, L)] = z
        return carry

      lax.fori_loop(0, E, zb, 0)

    def start_in(inb, sem, c):
      pltpu.async_copy(
          tab_hbm.at[pl.ds(0, E), pl.ds(c * CHUNK_C, CHUNK_C)], inb, sem)

    def wait_in(inb, sem):
      pltpu.make_async_copy(
          tab_hbm.at[pl.ds(0, E), pl.ds(0, CHUNK_C)], inb, sem).wait()

    def transpose(inb, ob, ncols):
      for r in range(ncols // 2):
        for g in range(8):
          col = 2 * r + (1 if g >= 4 else 0)
          vals = plsc.load_gather(
              inb, [row_vecs[g % 4], jnp.full((L,), col, jnp.int32)])
          ob[r, pl.ds(g * L, L)] = vals

    def start_out(ob, sem, row0, nrows):
      pltpu.async_copy(ob.at[pl.ds(0, nrows)],
                       pairs_hbm.at[pl.ds(row0, nrows)], sem)

    def wait_out(ob, sem, nrows):
      pltpu.make_async_copy(ob.at[pl.ds(0, nrows)],
                            pairs_hbm.at[pl.ds(ZROW, nrows)], sem).wait()

    zero_out(ob0)
    zero_out(ob1)
    start_out(ob0, so0, ZROW, E)
    start_out(ob1, so1, ZROW, E)
    start_in(in0, si0, wid)
    start_in(in1, si1, wid + 32)

    def body(kk, carry):
      c0 = wid + 64 * kk
      c1 = c0 + 32
      wait_in(in0, si0)
      wait_out(ob0, so0, E)
      transpose(in0, ob0, CHUNK_C)
      start_out(ob0, so0, c0 * (CHUNK_C // 2), E)

      @pl.when(c0 + 64 <= N_CHUNKS - 1)
      def _():
        start_in(in0, si0, c0 + 64)

      wait_in(in1, si1)
      wait_out(ob1, so1, E)
      transpose(in1, ob1, CHUNK_C)
      start_out(ob1, so1, c1 * (CHUNK_C // 2), E)

      @pl.when(c1 + 64 <= N_CHUNKS - 1)
      def _():
        start_in(in1, si1, c1 + 64)

      return carry

    lax.fori_loop(0, PAIR_LOOP, body, 0)

    @pl.when(wid <= 3)
    def _():
      c = wid + 64 * PAIR_LOOP
      wait_in(in0, si0)
      wait_out(ob0, so0, E)
      transpose(in0, ob0, CHUNK_C)
      start_out(ob0, so0, c * (CHUNK_C // 2), E)

    @pl.when(wid == 4)
    def _():
      pltpu.sync_copy(tail_hbm, in0.at[pl.ds(0, TAIL_C // 2)])
      wait_out(ob0, so0, E)
      pltpu.async_copy(in0.at[pl.ds(0, TAIL_C // 2)],
                       pairs_hbm.at[pl.ds(N_CHUNKS * (CHUNK_C // 2),
                                          TAIL_C // 2)], so0)

    @pl.when(wid <= 3)
    def _():
      wait_out(ob0, so0, E)

    @pl.when(wid == 4)
    def _():
      wait_out(ob0, so0, TAIL_C // 2)

    @pl.when(wid >= 5)
    def _():
      wait_out(ob0, so0, E)

    wait_out(ob1, so1, E)

  return k(tab_t, tail_pairs)


def _sc_pool(pairs, tids_t):
  mesh = plsc.VectorSubcoreMesh(
      core_axis_name="c", subcore_axis_name="s", num_cores=NC, num_subcores=NS)

  @functools.partial(
      pl.kernel,
      out_type=jax.ShapeDtypeStruct((B, E), jnp.float32),
      mesh=mesh,
      scratch_types=[
          pltpu.VMEM((S, SEQ_PER_W), jnp.int32),
          pltpu.VMEM((S, SEQ_PER_W), jnp.int32),
          pltpu.VMEM((S, SEQ_PER_W), jnp.int32),
          pltpu.VMEM((SEQ_PER_W, 128), jnp.float32),
          pltpu.VMEM((SEQ_PER_W, 128), jnp.float32),
          pltpu.VMEM((SEQ_PER_W, 128), jnp.float32),
          pltpu.VMEM((SEQ_PER_W, 128), jnp.float32),
          pltpu.VMEM((SEQ_PER_W, E), jnp.float32),
          pltpu.SemaphoreType.DMA,
          pltpu.SemaphoreType.DMA,
          pltpu.SemaphoreType.DMA,
          pltpu.SemaphoreType.DMA,
      ],
      compiler_params=pltpu.CompilerParams(needs_layout_passes=False),
  )
  def k(pairs_hbm, tid_hbm, out_hbm, idx_v, idx_e, idx_o,
        be0, be1, bo0, bo1, pooled_v, se0, se1, so0, so1):
    wid = lax.axis_index("s") * NC + lax.axis_index("c")
    base = wid * SEQ_PER_W
    pltpu.sync_copy(tid_hbm.at[pl.ds(0, S), pl.ds(base, SEQ_PER_W)], idx_v)

    one = jnp.full((L,), 1, jnp.int32)

    def prep_body(t, carry):
      for kk in range(SEQ_PER_W // L):
        v = idx_v[t, pl.ds(kk * L, L)]
        w = lax.shift_right_logical(v, one)
        p = lax.bitwise_and(v, one)
        z = jnp.full((L,), ZROW + kk, jnp.int32)
        idx_e[t, pl.ds(kk * L, L)] = jnp.where(p == 0, w, z)
        idx_o[t, pl.ds(kk * L, L)] = jnp.where(p == 1, w, z)
      return carry

    lax.fori_loop(0, S, prep_body, 0)

    def zero_body(j, carry):
      z = jnp.zeros((L,), jnp.float32)
      for d in range(E // L):
        pooled_v[j, pl.ds(d * L, L)] = z
      return carry

    lax.fori_loop(0, SEQ_PER_W, zero_body, 0)

    ebufs, obufs = [be0, be1], [bo0, bo1]
    esems, osems = [se0, se1], [so0, so1]

    def start(t):
      pltpu.async_copy(pairs_hbm.at[idx_e.at[t]], ebufs[t % 2], esems[t % 2])
      pltpu.async_copy(pairs_hbm.at[idx_o.at[t]], obufs[t % 2], osems[t % 2])

    def wait(t):
      pltpu.make_async_copy(pairs_hbm.at[idx_e.at[t]], ebufs[t % 2],
                            esems[t % 2]).wait()
      pltpu.make_async_copy(pairs_hbm.at[idx_o.at[t]], obufs[t % 2],
                            osems[t % 2]).wait()

    def accum(t):
      be, bo = ebufs[t % 2], obufs[t % 2]

      def acc_body(j, carry):
        for d in range(E // L):
          plsc.addupdate(pooled_v.at[j, pl.ds(d * L, L)],
                         be[j, pl.ds(d * L, L)] + bo[j, pl.ds(E + d * L, L)])
        return carry

      lax.fori_loop(0, SEQ_PER_W, acc_body, 0)

    start(0)
    start(1)
    for t in range(S):
      wait(t)
      accum(t)
      if t + 2 < S:
        start(t + 2)

    pltpu.sync_copy(pooled_v, out_hbm.at[pl.ds(base, SEQ_PER_W)])

  return k(pairs, tids_t)


def _tc_proj_body(x_ref, w_ref, b_ref, o_ref):
  x = x_ref[...] * jnp.float32(1.0 / S)
  o_ref[...] = jnp.tanh(
      jnp.dot(x, w_ref[...], preferred_element_type=jnp.float32) + b_ref[...])


def _tc_proj(pooled, W, b):
  blk = 512
  return pl.pallas_call(
      _tc_proj_body,
      grid=(B // blk,),
      in_specs=[
          pl.BlockSpec((blk, E), lambda i: (i, 0)),
          pl.BlockSpec((E, O), lambda i: (0, 0)),
          pl.BlockSpec((1, O), lambda i: (0, 0)),
      ],
      out_specs=pl.BlockSpec((blk, O), lambda i: (i, 0)),
      out_shape=jax.ShapeDtypeStruct((B, O), jnp.float32),
  )(pooled, W, b.reshape(1, O))


@jax.jit
def kernel(token_ids, table, W, b):
  tids_t = token_ids.astype(jnp.int32).T
  tail_pairs = table[V - TAIL_C:].reshape(TAIL_C // 2, 128)
  pairs = _sc_prep(table.T, tail_pairs)
  pooled = _sc_pool(pairs, tids_t)
  return _tc_proj(pooled, W, b)


# speedup vs baseline: 1.5446x; 1.5446x over previous
import functools

import jax
import jax.numpy as jnp
from jax import lax
from jax.experimental import pallas as pl
from jax.experimental.pallas import tpu as pltpu
from jax.experimental.pallas import tpu_sc as plsc

NC = 2
NS = 16
L = 16
NW = NC * NS

B = 4096
S = 50
E = 64
O = 128
V = 1000000

PAIR_ROWS = V // 2
ZPAD = 4096
PR = PAIR_ROWS + ZPAD
ZROW = PAIR_ROWS

CHUNK_C = 128
N_CHUNKS = V // CHUNK_C
TAIL_C = V - N_CHUNKS * CHUNK_C
PAIR_LOOP = 122

SEQ_PER_W = B // NW


def _sc_prep(tab_t, tail_pairs):
  mesh = plsc.VectorSubcoreMesh(
      core_axis_name="c", subcore_axis_name="s", num_cores=NC, num_subcores=NS)

  @functools.partial(
      pl.kernel,
      out_type=jax.ShapeDtypeStruct((PR, 128), jnp.float32),
      mesh=mesh,
      scratch_types=[
          pltpu.VMEM((E, CHUNK_C), jnp.float32),
          pltpu.VMEM((E, CHUNK_C), jnp.float32),
          pltpu.VMEM((E, 128), jnp.float32),
          pltpu.VMEM((E, 128), jnp.float32),
          pltpu.SemaphoreType.DMA,
          pltpu.SemaphoreType.DMA,
          pltpu.SemaphoreType.DMA,
          pltpu.SemaphoreType.DMA,
      ],
      compiler_params=pltpu.CompilerParams(needs_layout_passes=False),
  )
  def k(tab_hbm, tail_hbm, pairs_hbm, in0, in1, ob0, ob1, si0, si1, so0, so1):
    wid = lax.axis_index("s") * NC + lax.axis_index("c")

    row_vecs = [lax.iota(jnp.int32, L) + (g * L) for g in range(E // L)]

    def zero_out(ob):
      z = jnp.zeros((L,), jnp.float32)

      def zb(r, carry):
        for g in range(8):
          ob[r, pl.ds(g * L, L)] = z
        return carry

      lax.fori_loop(0, E, zb, 0)

    def start_in(inb, sem, c):
      pltpu.async_copy(
          tab_hbm.at[pl.ds(0, E), pl.ds(c * CHUNK_C, CHUNK_C)], inb, sem)

    def wait_in(inb, sem):
      pltpu.make_async_copy(
          tab_hbm.at[pl.ds(0, E), pl.ds(0, CHUNK_C)], inb, sem).wait()

    def transpose(inb, ob, ncols):
      for r in range(ncols // 2):
        for g in range(8):
          col = 2 * r + (1 if g >= 4 else 0)
          vals = plsc.load_gather(
              inb, [row_vecs[g % 4], jnp.full((L,), col, jnp.int32)])
          ob[r, pl.ds(g * L, L)] = vals

    def start_out(ob, sem, row0, nrows):
      pltpu.async_copy(ob.at[pl.ds(0, nrows)],
                       pairs_hbm.at[pl.ds(row0, nrows)], sem)

    def wait_out(ob, sem, nrows):
      pltpu.make_async_copy(ob.at[pl.ds(0, nrows)],
                            pairs_hbm.at[pl.ds(ZROW, nrows)], sem).wait()

    zero_out(ob0)
    zero_out(ob1)
    start_out(ob0, so0, ZROW + wid * 128, E)
    start_out(ob1, so1, ZROW + wid * 128 + E, E)
    start_in(in0, si0, wid)
    start_in(in1, si1, wid + 32)

    def body(kk, carry):
      c0 = wid + 64 * kk
      c1 = c0 + 32
      wait_in(in0, si0)
      wait_out(ob0, so0, E)
      transpose(in0, ob0, CHUNK_C)
      start_out(ob0, so0, c0 * (CHUNK_C // 2), E)

      @pl.when(c0 + 64 <= N_CHUNKS - 1)
      def _():
        start_in(in0, si0, c0 + 64)

      wait_in(in1, si1)
      wait_out(ob1, so1, E)
      transpose(in1, ob1, CHUNK_C)
      start_out(ob1, so1, c1 * (CHUNK_C // 2), E)

      @pl.when(c1 + 64 <= N_CHUNKS - 1)
      def _():
        start_in(in1, si1, c1 + 64)

      return carry

    lax.fori_loop(0, PAIR_LOOP, body, 0)

    @pl.when(wid <= 3)
    def _():
      c = wid + 64 * PAIR_LOOP
      wait_in(in0, si0)
      wait_out(ob0, so0, E)
      transpose(in0, ob0, CHUNK_C)
      start_out(ob0, so0, c * (CHUNK_C // 2), E)

    @pl.when(wid == 4)
    def _():
      pltpu.sync_copy(tail_hbm, in0.at[pl.ds(0, TAIL_C // 2)])
      wait_out(ob0, so0, E)
      pltpu.async_copy(in0.at[pl.ds(0, TAIL_C // 2)],
                       pairs_hbm.at[pl.ds(N_CHUNKS * (CHUNK_C // 2),
                                          TAIL_C // 2)], so0)

    @pl.when(wid <= 3)
    def _():
      wait_out(ob0, so0, E)

    @pl.when(wid == 4)
    def _():
      wait_out(ob0, so0, TAIL_C // 2)

    @pl.when(wid >= 5)
    def _():
      wait_out(ob0, so0, E)

    wait_out(ob1, so1, E)

  return k(tab_t, tail_pairs)


def _sc_pool(pairs, tids_t):
  mesh = plsc.VectorSubcoreMesh(
      core_axis_name="c", subcore_axis_name="s", num_cores=NC, num_subcores=NS)

  @functools.partial(
      pl.kernel,
      out_type=jax.ShapeDtypeStruct((B, E), jnp.float32),
      mesh=mesh,
      scratch_types=[
          pltpu.VMEM((S, SEQ_PER_W), jnp.int32),
          pltpu.VMEM((S, SEQ_PER_W), jnp.int32),
          pltpu.VMEM((S, SEQ_PER_W), jnp.int32),
          pltpu.VMEM((SEQ_PER_W, 128), jnp.float32),
          pltpu.VMEM((SEQ_PER_W, 128), jnp.float32),
          pltpu.VMEM((SEQ_PER_W, 128), jnp.float32),
          pltpu.VMEM((SEQ_PER_W, 128), jnp.float32),
          pltpu.VMEM((SEQ_PER_W, E), jnp.float32),
          pltpu.SemaphoreType.DMA,
          pltpu.SemaphoreType.DMA,
          pltpu.SemaphoreType.DMA,
          pltpu.SemaphoreType.DMA,
      ],
      compiler_params=pltpu.CompilerParams(needs_layout_passes=False),
  )
  def k(pairs_hbm, tid_hbm, out_hbm, idx_v, idx_e, idx_o,
        be0, be1, bo0, bo1, pooled_v, se0, se1, so0, so1):
    wid = lax.axis_index("s") * NC + lax.axis_index("c")
    base = wid * SEQ_PER_W
    pltpu.sync_copy(tid_hbm.at[pl.ds(0, S), pl.ds(base, SEQ_PER_W)], idx_v)

    one = jnp.full((L,), 1, jnp.int32)

    def prep_body(t, carry):
      for kk in range(SEQ_PER_W // L):
        v = idx_v[t, pl.ds(kk * L, L)]
        w = lax.shift_right_logical(v, one)
        p = lax.bitwise_and(v, one)
        zbase = ZROW + lax.rem(t * 128 + kk * L, ZPAD)
        z = lax.iota(jnp.int32, L) + zbase
        idx_e[t, pl.ds(kk * L, L)] = jnp.where(p == 0, w, z)
        idx_o[t, pl.ds(kk * L, L)] = jnp.where(p == 1, w, z)
      return carry

    lax.fori_loop(0, S, prep_body, 0)

    def zero_body(j, carry):
      z = jnp.zeros((L,), jnp.float32)
      for d in range(E // L):
        pooled_v[j, pl.ds(d * L, L)] = z
      return carry

    lax.fori_loop(0, SEQ_PER_W, zero_body, 0)

    ebufs, obufs = [be0, be1], [bo0, bo1]
    esems, osems = [se0, se1], [so0, so1]

    def start(t):
      pltpu.async_copy(pairs_hbm.at[idx_e.at[t]], ebufs[t % 2], esems[t % 2])
      pltpu.async_copy(pairs_hbm.at[idx_o.at[t]], obufs[t % 2], osems[t % 2])

    def wait(t):
      pltpu.make_async_copy(pairs_hbm.at[idx_e.at[t]], ebufs[t % 2],
                            esems[t % 2]).wait()
      pltpu.make_async_copy(pairs_hbm.at[idx_o.at[t]], obufs[t % 2],
                            osems[t % 2]).wait()

    def accum(t):
      be, bo = ebufs[t % 2], obufs[t % 2]

      def acc_body(j, carry):
        for d in range(E // L):
          plsc.addupdate(pooled_v.at[j, pl.ds(d * L, L)],
                         be[j, pl.ds(d * L, L)] + bo[j, pl.ds(E + d * L, L)])
        return carry

      lax.fori_loop(0, SEQ_PER_W, acc_body, 0)

    start(0)
    start(1)
    for t in range(S):
      wait(t)
      accum(t)
      if t + 2 < S:
        start(t + 2)

    pltpu.sync_copy(pooled_v, out_hbm.at[pl.ds(base, SEQ_PER_W)])

  return k(pairs, tids_t)


def _tc_proj_body(x_ref, w_ref, b_ref, o_ref):
  x = x_ref[...] * jnp.float32(1.0 / S)
  o_ref[...] = jnp.tanh(
      jnp.dot(x, w_ref[...], preferred_element_type=jnp.float32) + b_ref[...])


def _tc_proj(pooled, W, b):
  blk = 512
  return pl.pallas_call(
      _tc_proj_body,
      grid=(B // blk,),
      in_specs=[
          pl.BlockSpec((blk, E), lambda i: (i, 0)),
          pl.BlockSpec((E, O), lambda i: (0, 0)),
          pl.BlockSpec((1, O), lambda i: (0, 0)),
      ],
      out_specs=pl.BlockSpec((blk, O), lambda i: (i, 0)),
      out_shape=jax.ShapeDtypeStruct((B, O), jnp.float32),
  )(pooled, W, b.reshape(1, O))


@jax.jit
def kernel(token_ids, table, W, b):
  tids_t = token_ids.astype(jnp.int32).T
  tail_pairs = table[V - TAIL_C:].reshape(TAIL_C // 2, 128)
  pairs = _sc_prep(table.T, tail_pairs)
  pooled = _sc_pool(pairs, tids_t)
  return _tc_proj(pooled, W, b)


# speedup vs baseline: 2.8860x; 1.8685x over previous
import functools

import jax
import jax.numpy as jnp
from jax import lax
from jax.experimental import pallas as pl
from jax.experimental.pallas import tpu as pltpu
from jax.experimental.pallas import tpu_sc as plsc

NC = 2
NS = 16
L = 16
NW = NC * NS

B = 4096
S = 50
E = 64
O = 128
V = 1000000

PAIR_ROWS = V // 2
ZPAD = 4096
PR = PAIR_ROWS + ZPAD
ZROW = PAIR_ROWS

CHUNK_C = 128
N_CHUNKS = V // CHUNK_C
TAIL_C = V - N_CHUNKS * CHUNK_C
PAIR_LOOP = 122

SEQ_PER_W = B // NW


def _sc_prep(tab_t, tail_pairs):
  mesh = plsc.VectorSubcoreMesh(
      core_axis_name="c", subcore_axis_name="s", num_cores=NC, num_subcores=NS)

  @functools.partial(
      pl.kernel,
      out_type=jax.ShapeDtypeStruct((PR, 128), jnp.float32),
      mesh=mesh,
      scratch_types=[
          pltpu.VMEM((E, CHUNK_C), jnp.float32),
          pltpu.VMEM((E, CHUNK_C), jnp.float32),
          pltpu.VMEM((E, 128), jnp.float32),
          pltpu.VMEM((E, 128), jnp.float32),
          pltpu.SemaphoreType.DMA,
          pltpu.SemaphoreType.DMA,
          pltpu.SemaphoreType.DMA,
          pltpu.SemaphoreType.DMA,
      ],
      compiler_params=pltpu.CompilerParams(needs_layout_passes=False),
  )
  def k(tab_hbm, tail_hbm, pairs_hbm, in0, in1, ob0, ob1, si0, si1, so0, so1):
    wid = lax.axis_index("s") * NC + lax.axis_index("c")

    row_vecs = [lax.iota(jnp.int32, L) + (g * L) for g in range(E // L)]

    def zero_out(ob):
      z = jnp.zeros((L,), jnp.float32)

      def zb(r, carry):
        for g in range(8):
          ob[r, pl.ds(g * L, L)] = z
        return carry

      lax.fori_loop(0, E, zb, 0)

    def start_in(inb, sem, c):
      pltpu.async_copy(
          tab_hbm.at[pl.ds(0, E), pl.ds(c * CHUNK_C, CHUNK_C)], inb, sem)

    def wait_in(inb, sem):
      pltpu.make_async_copy(
          tab_hbm.at[pl.ds(0, E), pl.ds(0, CHUNK_C)], inb, sem).wait()

    def transpose(inb, ob, ncols):
      @plsc.parallel_loop(0, ncols // 2, unroll=4)
      def _(r):
        for g in range(8):
          col = 2 * r + (1 if g >= 4 else 0)
          vals = plsc.load_gather(
              inb, [row_vecs[g % 4], jnp.full((L,), col, jnp.int32)])
          ob[r, pl.ds(g * L, L)] = vals

    def start_out(ob, sem, row0, nrows):
      pltpu.async_copy(ob.at[pl.ds(0, nrows)],
                       pairs_hbm.at[pl.ds(row0, nrows)], sem)

    def wait_out(ob, sem, nrows):
      pltpu.make_async_copy(ob.at[pl.ds(0, nrows)],
                            pairs_hbm.at[pl.ds(ZROW, nrows)], sem).wait()

    zero_out(ob0)
    zero_out(ob1)
    start_out(ob0, so0, ZROW + wid * 128, E)
    start_out(ob1, so1, ZROW + wid * 128 + E, E)
    start_in(in0, si0, wid)
    start_in(in1, si1, wid + 32)

    def body(kk, carry):
      c0 = wid + 64 * kk
      c1 = c0 + 32
      wait_in(in0, si0)
      wait_out(ob0, so0, E)
      transpose(in0, ob0, CHUNK_C)
      start_out(ob0, so0, c0 * (CHUNK_C // 2), E)

      @pl.when(c0 + 64 <= N_CHUNKS - 1)
      def _():
        start_in(in0, si0, c0 + 64)

      wait_in(in1, si1)
      wait_out(ob1, so1, E)
      transpose(in1, ob1, CHUNK_C)
      start_out(ob1, so1, c1 * (CHUNK_C // 2), E)

      @pl.when(c1 + 64 <= N_CHUNKS - 1)
      def _():
        start_in(in1, si1, c1 + 64)

      return carry

    lax.fori_loop(0, PAIR_LOOP, body, 0)

    @pl.when(wid <= 3)
    def _():
      c = wid + 64 * PAIR_LOOP
      wait_in(in0, si0)
      wait_out(ob0, so0, E)
      transpose(in0, ob0, CHUNK_C)
      start_out(ob0, so0, c * (CHUNK_C // 2), E)

    @pl.when(wid == 4)
    def _():
      pltpu.sync_copy(tail_hbm, in0.at[pl.ds(0, TAIL_C // 2)])
      wait_out(ob0, so0, E)
      pltpu.async_copy(in0.at[pl.ds(0, TAIL_C // 2)],
                       pairs_hbm.at[pl.ds(N_CHUNKS * (CHUNK_C // 2),
                                          TAIL_C // 2)], so0)

    @pl.when(wid <= 3)
    def _():
      wait_out(ob0, so0, E)

    @pl.when(wid == 4)
    def _():
      wait_out(ob0, so0, TAIL_C // 2)

    @pl.when(wid >= 5)
    def _():
      wait_out(ob0, so0, E)

    wait_out(ob1, so1, E)

  return k(tab_t, tail_pairs)


def _sc_pool(pairs, tids_t):
  mesh = plsc.VectorSubcoreMesh(
      core_axis_name="c", subcore_axis_name="s", num_cores=NC, num_subcores=NS)

  @functools.partial(
      pl.kernel,
      out_type=jax.ShapeDtypeStruct((B, E), jnp.float32),
      mesh=mesh,
      scratch_types=[
          pltpu.VMEM((S, SEQ_PER_W), jnp.int32),
          pltpu.VMEM((S, SEQ_PER_W), jnp.int32),
          pltpu.VMEM((S, SEQ_PER_W), jnp.int32),
          pltpu.VMEM((SEQ_PER_W, 128), jnp.float32),
          pltpu.VMEM((SEQ_PER_W, 128), jnp.float32),
          pltpu.VMEM((SEQ_PER_W, 128), jnp.float32),
          pltpu.VMEM((SEQ_PER_W, 128), jnp.float32),
          pltpu.VMEM((SEQ_PER_W, E), jnp.float32),
          pltpu.SemaphoreType.DMA,
          pltpu.SemaphoreType.DMA,
          pltpu.SemaphoreType.DMA,
          pltpu.SemaphoreType.DMA,
      ],
      compiler_params=pltpu.CompilerParams(needs_layout_passes=False),
  )
  def k(pairs_hbm, tid_hbm, out_hbm, idx_v, idx_e, idx_o,
        be0, be1, bo0, bo1, pooled_v, se0, se1, so0, so1):
    wid = lax.axis_index("s") * NC + lax.axis_index("c")
    base = wid * SEQ_PER_W
    pltpu.sync_copy(tid_hbm.at[pl.ds(0, S), pl.ds(base, SEQ_PER_W)], idx_v)

    one = jnp.full((L,), 1, jnp.int32)

    def prep_body(t, carry):
      for kk in range(SEQ_PER_W // L):
        v = idx_v[t, pl.ds(kk * L, L)]
        w = lax.shift_right_logical(v, one)
        p = lax.bitwise_and(v, one)
        zbase = ZROW + lax.rem(t * 128 + kk * L, ZPAD)
        z = lax.iota(jnp.int32, L) + zbase
        idx_e[t, pl.ds(kk * L, L)] = jnp.where(p == 0, w, z)
        idx_o[t, pl.ds(kk * L, L)] = jnp.where(p == 1, w, z)
      return carry

    lax.fori_loop(0, S, prep_body, 0)

    def zero_body(j, carry):
      z = jnp.zeros((L,), jnp.float32)
      for d in range(E // L):
        pooled_v[j, pl.ds(d * L, L)] = z
      return carry

    lax.fori_loop(0, SEQ_PER_W, zero_body, 0)

    ebufs, obufs = [be0, be1], [bo0, bo1]
    esems, osems = [se0, se1], [so0, so1]

    def start(t, par):
      pltpu.async_copy(pairs_hbm.at[idx_e.at[t]], ebufs[par], esems[par])
      pltpu.async_copy(pairs_hbm.at[idx_o.at[t]], obufs[par], osems[par])

    def wait(par):
      pltpu.make_async_copy(pairs_hbm.at[idx_e.at[0]], ebufs[par],
                            esems[par]).wait()
      pltpu.make_async_copy(pairs_hbm.at[idx_o.at[0]], obufs[par],
                            osems[par]).wait()

    def accum(par):
      be, bo = ebufs[par], obufs[par]

      @plsc.parallel_loop(0, SEQ_PER_W, unroll=4)
      def _(j):
        for d in range(E // L):
          plsc.addupdate(pooled_v.at[j, pl.ds(d * L, L)],
                         be[j, pl.ds(d * L, L)] + bo[j, pl.ds(E + d * L, L)])

    def step(t, par):
      wait(par)
      accum(par)

      @pl.when(t + 2 < S)
      def _():
        start(t + 2, par)

    start(0, 0)
    start(1, 1)

    def tbody(tt, carry):
      step(2 * tt, 0)
      step(2 * tt + 1, 1)
      return carry

    lax.fori_loop(0, S // 2, tbody, 0)

    pltpu.sync_copy(pooled_v, out_hbm.at[pl.ds(base, SEQ_PER_W)])

  return k(pairs, tids_t)


def _tc_proj_body(x_ref, w_ref, b_ref, o_ref):
  x = x_ref[...] * jnp.float32(1.0 / S)
  o_ref[...] = jnp.tanh(
      jnp.dot(x, w_ref[...], preferred_element_type=jnp.float32) + b_ref[...])


def _tc_proj(pooled, W, b):
  blk = 512
  return pl.pallas_call(
      _tc_proj_body,
      grid=(B // blk,),
      in_specs=[
          pl.BlockSpec((blk, E), lambda i: (i, 0)),
          pl.BlockSpec((E, O), lambda i: (0, 0)),
          pl.BlockSpec((1, O), lambda i: (0, 0)),
      ],
      out_specs=pl.BlockSpec((blk, O), lambda i: (i, 0)),
      out_shape=jax.ShapeDtypeStruct((B, O), jnp.float32),
  )(pooled, W, b.reshape(1, O))


@jax.jit
def kernel(token_ids, table, W, b):
  tids_t = token_ids.astype(jnp.int32).T
  tail_pairs = table[V - TAIL_C:].reshape(TAIL_C // 2, 128)
  pairs = _sc_prep(table.T, tail_pairs)
  pooled = _sc_pool(pairs, tids_t)
  return _tc_proj(pooled, W, b)
